# baseline (speedup 1.0000x reference)
import functools

import jax
import jax.numpy as jnp
from jax import lax
from jax.experimental import pallas as pl
from jax.experimental.pallas import tpu as pltpu
from jax.experimental.pallas import tpu_sc as plsc

N = 10000
E = 320000
D = 128
C = 16

NC = 2
NS = 16
NW = NC * NS

NPAD = 10240
EPAD = 327680
EPT = EPAD // NW
CH = 128
NCH = EPT // CH
RPT = NPAD // NS

_f32 = jnp.float32

CH = 64
NCHT = EPT // CH
NCH0 = 256
NCH1 = 64
GDEPTH = 2


def _make_seg_sum(include_deg: bool):
    RING = 4
    LOOK = 2
    HROWS = 64
    mesh = plsc.VectorSubcoreMesh(
        core_axis_name="c", subcore_axis_name="s", num_cores=NC, num_subcores=NS
    )
    out_type = [jax.ShapeDtypeStruct((NC, NPAD, D), _f32)]
    scratch = [
        pltpu.VMEM((HROWS, 128), jnp.int32),
        pltpu.VMEM((RING, CH), jnp.int32),
        pltpu.VMEM((RING, CH), jnp.int32),
        pltpu.VMEM((RING, CH, D), _f32),
        pltpu.VMEM((8, D), _f32),
        pltpu.VMEM_SHARED((NPAD, D), _f32),
        [pltpu.SemaphoreType.DMA] * RING,
        [pltpu.SemaphoreType.DMA] * RING,
        pltpu.SemaphoreType.DMA,
    ]
    if include_deg:
        out_type.append(jax.ShapeDtypeStruct((NC, NPAD), _f32))
        scratch += [
            pltpu.VMEM((CH,), _f32),
            pltpu.VMEM((RPT,), _f32),
            pltpu.VMEM_SHARED((NPAD,), _f32),
            [pltpu.SemaphoreType.DMA] * RING,
        ]

    def body(comb2d, feat, *rest):
        if include_deg:
            (agg_out, deg_out, comb, srcs, dsts, rows, zblk, acc, gsems,
             ssems, sem_z, ones_v, dzero, dacc, dsems) = rest
        else:
            (agg_out, comb, srcs, dsts, rows, zblk, acc, gsems, ssems,
             sem_z) = rest

        c = lax.axis_index("c")
        s = lax.axis_index("s")
        nch = jnp.where(c == 0, NCH0, NCH1)
        rb = jnp.where(c == 0, s * (NCH0 // 2),
                       NS * (NCH0 // 2) + s * (NCH1 // 2))

        for i in range(8):
            for j in range(D // 16):
                zblk[i, pl.ds(j * 16, 16)] = jnp.zeros((16,), _f32)
        if include_deg:
            for j in range(CH // 16):
                ones_v[pl.ds(j * 16, 16)] = jnp.ones((16,), _f32)

            def dzfill(t, carry):
                dzero[pl.ds(t * 16, 16)] = jnp.zeros((16,), _f32)
                return carry

            lax.fori_loop(0, RPT // 16, dzfill, 0)

        base = s * RPT
        zcps = [
            pltpu.async_copy(zblk, acc.at[pl.ds(base + t * 8, 8)], sem_z)
            for t in range(RPT // 8)
        ]
        pltpu.sync_copy(comb2d.at[pl.ds(rb, HROWS)], comb)
        for cp in zcps:
            cp.wait()
        if include_deg:
            pltpu.sync_copy(dzero, dacc.at[pl.ds(base, RPT)])
        plsc.subcore_barrier()

        def unpack(j, slot):
            row = (j % (2 * HROWS)) // 2
            half = (j % 2) * CH
            for t in range(CH // 16):
                v = comb[row, pl.ds(half + t * 16, 16)]
                srcs[slot, pl.ds(t * 16, 16)] = jnp.bitwise_and(v, 0xFFFF)
                dsts[slot, pl.ds(t * 16, 16)] = jnp.right_shift(v, 16)

        def gather(slot):
            pltpu.async_copy(feat.at[srcs.at[slot]], rows.at[slot],
                             gsems[slot])

        def wait_g(slot):
            pltpu.make_async_copy(feat.at[srcs.at[slot]], rows.at[slot],
                                  gsems[slot]).wait()

        def scatter(slot):
            pltpu.async_copy(rows.at[slot], acc.at[dsts.at[slot]],
                             ssems[slot], add=True)
            if include_deg:
                pltpu.async_copy(ones_v, dacc.at[dsts.at[slot]],
                                 dsems[slot], add=True)

        def wait_s(slot):
            pltpu.make_async_copy(rows.at[slot], acc.at[dsts.at[slot]],
                                  ssems[slot]).wait()
            if include_deg:
                pltpu.make_async_copy(ones_v, dacc.at[dsts.at[slot]],
                                      dsems[slot]).wait()

        for b in range(LOOK):
            unpack(b, b)
            gather(b)

        def eloop(jj, carry):
            j = RING * jj

            @pl.when(j < nch)
            def _():
                for b in range(RING):
                    jb = j + b
                    wait_g(b)
                    scatter(b)

                    @pl.when(jb >= LOOK)
                    def _():
                        wait_s((b + LOOK) % RING)

                    unpack(jnp.minimum(jb + LOOK, nch - 1), (b + LOOK) % RING)
                    gather((b + LOOK) % RING)

                    @pl.when(jb == 2 * HROWS - LOOK - 1)
                    def _():
                        pltpu.sync_copy(comb2d.at[pl.ds(rb + HROWS, HROWS)],
                                        comb)

            return carry

        lax.fori_loop(0, NCH0 // RING, eloop, 0)
        for b in range(LOOK):
            wait_g(b)
        for b in range(LOOK, RING):
            wait_s(b)
        plsc.subcore_barrier()

        pltpu.sync_copy(acc.at[pl.ds(base, RPT)],
                        agg_out.at[c, pl.ds(base, RPT)])
        if include_deg:
            pltpu.sync_copy(dacc.at[pl.ds(base, RPT)],
                            deg_out.at[c, pl.ds(base, RPT)])

    return pl.kernel(body, out_type=out_type, mesh=mesh, scratch_types=scratch)


_seg_sum_deg = _make_seg_sum(True)
_seg_sum = _make_seg_sum(False)


def _leaky(v):
    return jnp.where(v >= 0, v, 0.01 * v)


def _dot(a, b):
    return jax.lax.dot_general(
        a, b, (((1,), (0,)), ((), ())),
        precision=jax.lax.Precision.HIGHEST,
        preferred_element_type=_f32,
    )


_R = 2000


def _l0_body(x_ref, aggp_ref, degt_ref, ws_ref, wn_ref, b_ref, y_ref):
    agg = aggp_ref[0] + aggp_ref[1]
    deg = degt_ref[:, 0:1] + degt_ref[:, 1:2]
    hn = agg / jnp.maximum(deg, 1.0)
    v = _dot(x_ref[...], ws_ref[...]) + _dot(hn, wn_ref[...]) + b_ref[...]
    y_ref[...] = _leaky(v)


@jax.jit
def _tc_layer0(x, aggp, degt, ws, wn, b):
    return pl.pallas_call(
        _l0_body,
        grid=(N // _R,),
        in_specs=[
            pl.BlockSpec((_R, D), lambda i: (i, 0)),
            pl.BlockSpec((NC, _R, D), lambda i: (0, i, 0)),
            pl.BlockSpec((_R, NC), lambda i: (i, 0)),
            pl.BlockSpec((D, D), lambda i: (0, 0)),
            pl.BlockSpec((D, D), lambda i: (0, 0)),
            pl.BlockSpec((1, D), lambda i: (0, 0)),
        ],
        out_specs=pl.BlockSpec((_R, D), lambda i: (i, 0)),
        out_shape=jax.ShapeDtypeStruct((N, D), _f32),
    )(x, aggp, degt, ws, wn, b)


def _l1_body(y_ref, aggp_ref, degt_ref, ws_ref, wn_ref, b_ref,
             wc1_ref, bc1_ref, wc2_ref, bc2_ref, out_ref, em_ref):
    i = pl.program_id(0)
    agg = aggp_ref[0] + aggp_ref[1]
    deg = degt_ref[:, 0:1] + degt_ref[:, 1:2]
    hn = agg / jnp.maximum(deg, 1.0)
    y1 = _leaky(_dot(y_ref[...], ws_ref[...]) + _dot(hn, wn_ref[...])
                + b_ref[...])
    ssum = jnp.sum(y1, axis=0, keepdims=True)

    @pl.when(i == 0)
    def _():
        em_ref[...] = ssum

    @pl.when(i > 0)
    def _():
        em_ref[...] = em_ref[...] + ssum

    @pl.when(i == N // _R - 1)
    def _():
        h = _leaky(_dot(em_ref[...], wc1_ref[...]) + bc1_ref[...])
        out_ref[...] = _dot(h, wc2_ref[...]) + bc2_ref[...]


@jax.jit
def _tc_layer1_cls(y0, aggp, degt, ws, wn, b, wc1, bc1, wc2, bc2):
    return pl.pallas_call(
        _l1_body,
        grid=(N // _R,),
        in_specs=[
            pl.BlockSpec((_R, D), lambda i: (i, 0)),
            pl.BlockSpec((NC, _R, D), lambda i: (0, i, 0)),
            pl.BlockSpec((_R, NC), lambda i: (i, 0)),
            pl.BlockSpec((D, D), lambda i: (0, 0)),
            pl.BlockSpec((D, D), lambda i: (0, 0)),
            pl.BlockSpec((1, D), lambda i: (0, 0)),
            pl.BlockSpec((D, D), lambda i: (0, 0)),
            pl.BlockSpec((1, D), lambda i: (0, 0)),
            pl.BlockSpec((D, C), lambda i: (0, 0)),
            pl.BlockSpec((1, C), lambda i: (0, 0)),
        ],
        out_specs=pl.BlockSpec((1, C), lambda i: (0, 0)),
        out_shape=jax.ShapeDtypeStruct((1, C), _f32),
        scratch_shapes=[pltpu.VMEM((1, D), _f32)],
    )(y0, aggp, degt, ws, wn, b, wc1, bc1, wc2, bc2)


def kernel(x, edge_index, W_self0, W_neigh0, b0, W_self1, W_neigh1, b1,
           W_cls1, b_cls1, W_cls2, b_cls2):
    src = edge_index[0]
    dst = edge_index[1]
    pad = EPAD - E
    srcp = jnp.concatenate([src, jnp.zeros((pad,), jnp.int32)])
    dstp = jnp.concatenate([dst, jnp.full((pad,), N, jnp.int32)])
    comb2d = jnp.bitwise_or(srcp, jnp.left_shift(dstp, 16)).reshape(
        EPAD // 128, 128)
    comb2d = jnp.concatenate(
        [comb2d, jnp.zeros((NCH0 // 2, 128), jnp.int32)], axis=0)

    agg0p, degp = _seg_sum_deg(comb2d, x)
    degt = degp.T
    y0 = _tc_layer0(x, agg0p, degt, W_self0, W_neigh0, b0.reshape(1, D))
    (agg1p,) = _seg_sum(comb2d, y0)
    out = _tc_layer1_cls(
        y0, agg1p, degt, W_self1, W_neigh1, b1.reshape(1, D),
        W_cls1, b_cls1.reshape(1, D), W_cls2, b_cls2.reshape(1, C),
    )
    return out


# speedup vs baseline: 1.1863x; 1.1863x over previous
import functools

import jax
import jax.numpy as jnp
from jax import lax
from jax.experimental import pallas as pl
from jax.experimental.pallas import tpu as pltpu
from jax.experimental.pallas import tpu_sc as plsc

N = 10000
E = 320000
D = 128
C = 16

NC = 2
NS = 16
NW = NC * NS

NPAD = 10240
EPAD = 327680
EPT = EPAD // NW
CH = 128
NCH = EPT // CH
RPT = NPAD // NS

_f32 = jnp.float32

CH = 128
NCH0 = 128
NCH1 = 32


def _make_seg_sum(include_deg: bool):
    RING = 2
    LOOK = 1
    HROWS = 64
    mesh = plsc.VectorSubcoreMesh(
        core_axis_name="c", subcore_axis_name="s", num_cores=NC, num_subcores=NS
    )
    out_type = [jax.ShapeDtypeStruct((NC, NPAD, D), _f32)]
    scratch = [
        pltpu.VMEM((HROWS, 128), jnp.int32),
        pltpu.VMEM((RING, CH), jnp.int32),
        pltpu.VMEM((RING, CH), jnp.int32),
        pltpu.VMEM((RING, CH, D), _f32),
        pltpu.VMEM((8, D), _f32),
        pltpu.VMEM_SHARED((NPAD, D), _f32),
        [pltpu.SemaphoreType.DMA] * RING,
        [pltpu.SemaphoreType.DMA] * RING,
        pltpu.SemaphoreType.DMA,
    ]
    if include_deg:
        out_type.append(jax.ShapeDtypeStruct((NC, NPAD), _f32))
        scratch += [
            pltpu.VMEM((CH,), _f32),
            pltpu.VMEM((RPT,), _f32),
            pltpu.VMEM_SHARED((NPAD,), _f32),
            [pltpu.SemaphoreType.DMA] * RING,
        ]

    def body(comb2d, feat, *rest):
        if include_deg:
            (agg_out, deg_out, comb, srcs, dsts, rows, zblk, acc, gsems,
             ssems, sem_z, ones_v, dzero, dacc, dsems) = rest
        else:
            (agg_out, comb, srcs, dsts, rows, zblk, acc, gsems, ssems,
             sem_z) = rest

        c = lax.axis_index("c")
        s = lax.axis_index("s")
        nch = jnp.where(c == 0, NCH0, NCH1)
        rb = jnp.where(c == 0, s * NCH0, NS * NCH0 + s * NCH1)

        for i in range(8):
            for j in range(D // 16):
                zblk[i, pl.ds(j * 16, 16)] = jnp.zeros((16,), _f32)
        if include_deg:
            for j in range(CH // 16):
                ones_v[pl.ds(j * 16, 16)] = jnp.ones((16,), _f32)

            def dzfill(t, carry):
                dzero[pl.ds(t * 16, 16)] = jnp.zeros((16,), _f32)
                return carry

            lax.fori_loop(0, RPT // 16, dzfill, 0)

        base = s * RPT
        zcps = [
            pltpu.async_copy(zblk, acc.at[pl.ds(base + t * 8, 8)], sem_z)
            for t in range(RPT // 8)
        ]
        pltpu.sync_copy(comb2d.at[pl.ds(rb, HROWS)], comb)
        for cp in zcps:
            cp.wait()
        if include_deg:
            pltpu.sync_copy(dzero, dacc.at[pl.ds(base, RPT)])
        plsc.subcore_barrier()

        def unpack(j, slot):
            row = j % HROWS
            for t in range(CH // 16):
                v = comb[row, pl.ds(t * 16, 16)]
                srcs[slot, pl.ds(t * 16, 16)] = jnp.bitwise_and(v, 0xFFFF)
                dsts[slot, pl.ds(t * 16, 16)] = jnp.right_shift(v, 16)

        def gather(slot):
            pltpu.async_copy(feat.at[srcs.at[slot]], rows.at[slot],
                             gsems[slot])

        def wait_g(slot):
            pltpu.make_async_copy(feat.at[srcs.at[slot]], rows.at[slot],
                                  gsems[slot]).wait()

        def scatter(slot):
            pltpu.async_copy(rows.at[slot], acc.at[dsts.at[slot]],
                             ssems[slot], add=True)
            if include_deg:
                pltpu.async_copy(ones_v, dacc.at[dsts.at[slot]],
                                 dsems[slot], add=True)

        def wait_s(slot):
            pltpu.make_async_copy(rows.at[slot], acc.at[dsts.at[slot]],
                                  ssems[slot]).wait()
            if include_deg:
                pltpu.make_async_copy(ones_v, dacc.at[dsts.at[slot]],
                                      dsems[slot]).wait()

        for b in range(LOOK):
            unpack(b, b)
            gather(b)

        def eloop(jj, carry):
            j = RING * jj

            @pl.when(j < nch)
            def _():
                for b in range(RING):
                    jb = j + b
                    wait_g(b)
                    scatter(b)

                    @pl.when(jb >= LOOK)
                    def _():
                        wait_s((b + LOOK) % RING)

                    unpack(jnp.minimum(jb + LOOK, nch - 1), (b + LOOK) % RING)
                    gather((b + LOOK) % RING)

                    @pl.when(jb == HROWS - LOOK - 1)
                    def _():
                        pltpu.sync_copy(comb2d.at[pl.ds(rb + HROWS, HROWS)],
                                        comb)

            return carry

        lax.fori_loop(0, NCH0 // RING, eloop, 0)
        for b in range(LOOK):
            wait_g(b)
        for b in range(LOOK, RING):
            wait_s(b)
        plsc.subcore_barrier()

        pltpu.sync_copy(acc.at[pl.ds(base, RPT)],
                        agg_out.at[c, pl.ds(base, RPT)])
        if include_deg:
            pltpu.sync_copy(dacc.at[pl.ds(base, RPT)],
                            deg_out.at[c, pl.ds(base, RPT)])

    return pl.kernel(body, out_type=out_type, mesh=mesh, scratch_types=scratch)


_seg_sum_deg = _make_seg_sum(True)
_seg_sum = _make_seg_sum(False)


def _leaky(v):
    return jnp.where(v >= 0, v, 0.01 * v)


def _dot(a, b):
    return jax.lax.dot_general(
        a, b, (((1,), (0,)), ((), ())),
        precision=jax.lax.Precision.HIGHEST,
        preferred_element_type=_f32,
    )


_R = 2000


def _l0_body(x_ref, aggp_ref, degt_ref, ws_ref, wn_ref, b_ref, y_ref):
    agg = aggp_ref[0] + aggp_ref[1]
    deg = degt_ref[:, 0:1] + degt_ref[:, 1:2]
    hn = agg / jnp.maximum(deg, 1.0)
    v = _dot(x_ref[...], ws_ref[...]) + _dot(hn, wn_ref[...]) + b_ref[...]
    y_ref[...] = _leaky(v)


@jax.jit
def _tc_layer0(x, aggp, degt, ws, wn, b):
    return pl.pallas_call(
        _l0_body,
        grid=(N // _R,),
        in_specs=[
            pl.BlockSpec((_R, D), lambda i: (i, 0)),
            pl.BlockSpec((NC, _R, D), lambda i: (0, i, 0)),
            pl.BlockSpec((_R, NC), lambda i: (i, 0)),
            pl.BlockSpec((D, D), lambda i: (0, 0)),
            pl.BlockSpec((D, D), lambda i: (0, 0)),
            pl.BlockSpec((1, D), lambda i: (0, 0)),
        ],
        out_specs=pl.BlockSpec((_R, D), lambda i: (i, 0)),
        out_shape=jax.ShapeDtypeStruct((N, D), _f32),
    )(x, aggp, degt, ws, wn, b)


def _l1_body(y_ref, aggp_ref, degt_ref, ws_ref, wn_ref, b_ref,
             wc1_ref, bc1_ref, wc2_ref, bc2_ref, out_ref, em_ref):
    i = pl.program_id(0)
    agg = aggp_ref[0] + aggp_ref[1]
    deg = degt_ref[:, 0:1] + degt_ref[:, 1:2]
    hn = agg / jnp.maximum(deg, 1.0)
    y1 = _leaky(_dot(y_ref[...], ws_ref[...]) + _dot(hn, wn_ref[...])
                + b_ref[...])
    ssum = jnp.sum(y1, axis=0, keepdims=True)

    @pl.when(i == 0)
    def _():
        em_ref[...] = ssum

    @pl.when(i > 0)
    def _():
        em_ref[...] = em_ref[...] + ssum

    @pl.when(i == N // _R - 1)
    def _():
        h = _leaky(_dot(em_ref[...], wc1_ref[...]) + bc1_ref[...])
        out_ref[...] = _dot(h, wc2_ref[...]) + bc2_ref[...]


@jax.jit
def _tc_layer1_cls(y0, aggp, degt, ws, wn, b, wc1, bc1, wc2, bc2):
    return pl.pallas_call(
        _l1_body,
        grid=(N // _R,),
        in_specs=[
            pl.BlockSpec((_R, D), lambda i: (i, 0)),
            pl.BlockSpec((NC, _R, D), lambda i: (0, i, 0)),
            pl.BlockSpec((_R, NC), lambda i: (i, 0)),
            pl.BlockSpec((D, D), lambda i: (0, 0)),
            pl.BlockSpec((D, D), lambda i: (0, 0)),
            pl.BlockSpec((1, D), lambda i: (0, 0)),
            pl.BlockSpec((D, D), lambda i: (0, 0)),
            pl.BlockSpec((1, D), lambda i: (0, 0)),
            pl.BlockSpec((D, C), lambda i: (0, 0)),
            pl.BlockSpec((1, C), lambda i: (0, 0)),
        ],
        out_specs=pl.BlockSpec((1, C), lambda i: (0, 0)),
        out_shape=jax.ShapeDtypeStruct((1, C), _f32),
        scratch_shapes=[pltpu.VMEM((1, D), _f32)],
    )(y0, aggp, degt, ws, wn, b, wc1, bc1, wc2, bc2)


def kernel(x, edge_index, W_self0, W_neigh0, b0, W_self1, W_neigh1, b1,
           W_cls1, b_cls1, W_cls2, b_cls2):
    src = edge_index[0]
    dst = edge_index[1]
    pad = EPAD - E
    srcp = jnp.concatenate([src, jnp.zeros((pad,), jnp.int32)])
    dstp = jnp.concatenate([dst, jnp.full((pad,), N, jnp.int32)])
    comb2d = jnp.bitwise_or(srcp, jnp.left_shift(dstp, 16)).reshape(
        EPAD // 128, 128)
    comb2d = jnp.concatenate(
        [comb2d, jnp.zeros((NCH0 // 2, 128), jnp.int32)], axis=0)

    agg0p, degp = _seg_sum_deg(comb2d, x)
    degt = degp.T
    y0 = _tc_layer0(x, agg0p, degt, W_self0, W_neigh0, b0.reshape(1, D))
    (agg1p,) = _seg_sum(comb2d, y0)
    out = _tc_layer1_cls(
        y0, agg1p, degt, W_self1, W_neigh1, b1.reshape(1, D),
        W_cls1, b_cls1.reshape(1, D), W_cls2, b_cls2.reshape(1, C),
    )
    return out
